# Initial kernel scaffold; baseline (speedup 1.0000x reference)
import jax
import jax.numpy as jnp
from jax.experimental import pallas as pl


def kernel(x, edge_index, emb, W1, a_src1, a_dst1, b1, W2, a_src2, a_dst2, b2):
    raise NotImplementedError("write your pallas kernel here")


# speedup vs baseline: 13.8255x; 13.8255x over previous
import functools

import jax
import jax.numpy as jnp
from jax import lax
from jax.experimental import pallas as pl
from jax.experimental.pallas import tpu as pltpu
from jax.experimental.pallas import tpu_sc as plsc

N = 10000
E = 160000
C = 128
H = 8
HID = 128
DM = 256
NP = 10240
NB = 1024
NSC = 2
NT = 16
PW = 128

_MESH = plsc.VectorSubcoreMesh(core_axis_name="c", subcore_axis_name="s")


def _f32(x):
    return x.astype(jnp.float32)


EPT1 = E // (NSC * NT)
_HROWS = NP * C
_HSTRIPE = _HROWS // NT
_ZW1 = 4096


@functools.partial(
    pl.kernel,
    out_type=jax.ShapeDtypeStruct((NSC, _HROWS), jnp.float32),
    mesh=_MESH,
    compiler_params=pltpu.CompilerParams(needs_layout_passes=False),
    scratch_types=[
        pltpu.VMEM((N,), jnp.int32),
        pltpu.VMEM((EPT1,), jnp.int32),
        pltpu.VMEM((EPT1,), jnp.int32),
        pltpu.VMEM((128,), jnp.int32),
        pltpu.VMEM((128,), jnp.float32),
        pltpu.VMEM((16,), jnp.int32),
        pltpu.VMEM((16,), jnp.float32),
        pltpu.VMEM((_ZW1,), jnp.float32),
        pltpu.VMEM_SHARED((_HROWS,), jnp.float32),
    ],
)
def _sc_hist(x_hbm, src_hbm, dst_hbm, out_hbm, xv, srcv, dstv, keyb, valb,
             keyt, valt, zb, csp):
    c = lax.axis_index("c")
    s = lax.axis_index("s")
    est = (c * NT + s) * EPT1
    pltpu.sync_copy(src_hbm.at[pl.ds(est, EPT1)], srcv)
    pltpu.sync_copy(dst_hbm.at[pl.ds(est, EPT1)], dstv)
    pltpu.sync_copy(x_hbm, xv)

    zero16 = jnp.zeros((16,), jnp.float32)
    one16 = jnp.full((16,), 1.0, jnp.float32)

    def zfill(i, _):
        zb[pl.ds(i * 16, 16)] = zero16
        return 0
    lax.fori_loop(0, _ZW1 // 16, zfill, 0)

    def ofill(i, _):
        valb[pl.ds(i * 16, 16)] = one16
        return 0
    lax.fori_loop(0, 8, ofill, 0)

    def zcopy(i, _):
        pltpu.sync_copy(zb, csp.at[pl.ds(s * _HSTRIPE + i * _ZW1, _ZW1)])
        return 0
    lax.fori_loop(0, _HSTRIPE // _ZW1, zcopy, 0)
    plsc.subcore_barrier()

    def chunk(ch, _):
        def grp(g, _):
            off = ch * 128 + g * 16
            s16 = srcv[pl.ds(off, 16)]
            d16 = dstv[pl.ds(off, 16)]
            cls = plsc.load_gather(xv, [s16])
            keyb[pl.ds(g * 16, 16)] = d16 * C + cls
            return 0
        lax.fori_loop(0, 8, grp, 0)
        pltpu.sync_copy(valb, csp.at[keyb], add=True)
        return 0
    lax.fori_loop(0, 39, chunk, 0)

    off = EPT1 - 16
    s16 = srcv[pl.ds(off, 16)]
    d16 = dstv[pl.ds(off, 16)]
    cls = plsc.load_gather(xv, [s16])
    lane = lax.iota(jnp.int32, 16)
    keyt[...] = jnp.where(lane >= 8, d16 * C + cls, 0)
    valt[...] = jnp.where(lane >= 8, 1.0, 0.0)
    pltpu.sync_copy(valt, csp.at[keyt], add=True)

    plsc.subcore_barrier()
    pltpu.sync_copy(csp.at[pl.ds(s * _HSTRIPE, _HSTRIPE)],
                    out_hbm.at[c, pl.ds(s * _HSTRIPE, _HSTRIPE)])


EPT2 = E // NT
HP = H // NSC
_K2 = 80
_NCH = EPT2 // _K2
_RPT = NP // NT
_ZR2 = 32
_RPTD = NP // NT


@functools.partial(
    pl.kernel,
    out_type=[jax.ShapeDtypeStruct((H * NP, PW), jnp.float32),
              jax.ShapeDtypeStruct((H * NP,), jnp.float32)],
    mesh=_MESH,
    compiler_params=pltpu.CompilerParams(needs_layout_passes=False),
    scratch_types=[
        pltpu.VMEM((N,), jnp.float32),
        pltpu.VMEM((N,), jnp.float32),
        pltpu.VMEM((_K2,), jnp.int32),
        pltpu.VMEM((_K2,), jnp.int32),
        pltpu.VMEM((_K2,), jnp.int32),
        pltpu.VMEM((_K2,), jnp.float32),
        pltpu.VMEM((_K2, 128), jnp.float32),
        pltpu.VMEM((_K2, PW), jnp.float32),
        pltpu.VMEM((_ZR2, PW), jnp.float32),
        pltpu.VMEM((_RPTD,), jnp.float32),
        pltpu.VMEM_SHARED((NP, PW), jnp.float32),
        pltpu.VMEM_SHARED((NP,), jnp.float32),
        pltpu.SemaphoreType.DMA,
    ],
)
def _sc_l2(src_hbm, dst_hbm, g2_hbm, asrc_hbm, adst_hbm, out_hbm, den_hbm,
           av, bv, srcb, idxb, dstb, exb, grow, sb, zb, zbd, psp, dsp, sem):
    c = lax.axis_index("c")
    s = lax.axis_index("s")
    est = s * EPT2

    zero16 = jnp.zeros((16,), jnp.float32)

    def zfill_row(i, _):
        def zcol(j, _):
            zb[i, pl.ds(j * 16, 16)] = zero16
            return 0
        lax.fori_loop(0, PW // 16, zcol, 0)
        return 0
    lax.fori_loop(0, _ZR2, zfill_row, 0)

    def zfill_d(i, _):
        zbd[pl.ds(i * 16, 16)] = zero16
        return 0
    lax.fori_loop(0, _RPTD // 16, zfill_d, 0)

    for p in range(HP):
        h = c * HP + p
        pltpu.sync_copy(asrc_hbm.at[pl.ds(h * NP, N)], av)
        pltpu.sync_copy(adst_hbm.at[pl.ds(h * NP, N)], bv)

        def zcopy(i, _):
            pltpu.sync_copy(zb, psp.at[pl.ds(s * _RPT + i * _ZR2, _ZR2)])
            return 0
        lax.fori_loop(0, _RPT // _ZR2, zcopy, 0)
        pltpu.sync_copy(zbd, dsp.at[pl.ds(s * _RPTD, _RPTD)])
        plsc.subcore_barrier()

        def chunk(ch, _):
            base = est + ch * _K2
            pltpu.sync_copy(src_hbm.at[pl.ds(base, _K2)], srcb)
            pltpu.sync_copy(dst_hbm.at[pl.ds(base, _K2)], dstb)

            def grp(g, _):
                s16 = srcb[pl.ds(g * 16, 16)]
                d16 = dstb[pl.ds(g * 16, 16)]
                idxb[pl.ds(g * 16, 16)] = s16 + h * NP
                e = plsc.load_gather(av, [s16]) + plsc.load_gather(bv, [d16])
                e = jnp.where(e >= 0.0, e, 0.2 * e)
                exb[pl.ds(g * 16, 16)] = jnp.exp(e)
                return 0
            lax.fori_loop(0, _K2 // 16, grp, 0)

            pltpu.async_copy(g2_hbm.at[idxb], grow, sem).wait()

            def edge(i, _):
                exs = plsc.load_gather(exb, [jnp.full((16,), i, jnp.int32)])
                for j in range(8):
                    sb[i, pl.ds(j * 16, 16)] = grow[i, pl.ds(j * 16, 16)] * exs
                return 0
            lax.fori_loop(0, _K2, edge, 0)

            pltpu.sync_copy(sb, psp.at[dstb], add=True)
            pltpu.sync_copy(exb, dsp.at[dstb], add=True)
            return 0
        lax.fori_loop(0, _NCH, chunk, 0)
        plsc.subcore_barrier()

        pltpu.sync_copy(psp.at[pl.ds(s * _RPT, _RPT)],
                        out_hbm.at[pl.ds(h * NP + s * _RPT, _RPT)])
        pltpu.sync_copy(dsp.at[pl.ds(s * _RPTD, _RPTD)],
                        den_hbm.at[pl.ds(h * NP + s * _RPTD, _RPTD)])


def _tc_tables_body(emb_ref, w1_ref, as1_ref, ad1_ref, w2_ref, as2_ref,
                    ad2_ref, ftab_ref, hw1_ref, vs2_ref, vd2_ref):
    hw = jnp.dot(emb_ref[...], w1_ref[...], preferred_element_type=jnp.float32)
    hw1_ref[...] = hw
    dn = (((1,), (1,)), ((), ()))
    for h in range(H):
        hwh = hw[:, h * HID:(h + 1) * HID]
        a1c = lax.dot_general(hwh, as1_ref[h:h + 1, :], dn,
                              preferred_element_type=jnp.float32)
        b1r = lax.dot_general(ad1_ref[h:h + 1, :], hwh, dn,
                              preferred_element_type=jnp.float32)
        e = a1c + b1r
        ftab_ref[h] = jnp.exp(jnp.where(e >= 0.0, e, 0.2 * e))
        w2h = w2_ref[:, h * C:(h + 1) * C]
        vs2_ref[:, h:h + 1] = lax.dot_general(
            w2h, as2_ref[h:h + 1, :], dn, preferred_element_type=jnp.float32)
        vd2_ref[:, h:h + 1] = lax.dot_general(
            w2h, ad2_ref[h:h + 1, :], dn, preferred_element_type=jnp.float32)


def _tc_tables(emb, W1, a_src1, a_dst1, W2, a_src2, a_dst2):
    return pl.pallas_call(
        _tc_tables_body,
        out_shape=[
            jax.ShapeDtypeStruct((H, C, C), jnp.float32),
            jax.ShapeDtypeStruct((C, H * HID), jnp.float32),
            jax.ShapeDtypeStruct((HID, H), jnp.float32),
            jax.ShapeDtypeStruct((HID, H), jnp.float32),
        ],
    )(emb, W1, a_src1, a_dst1, W2, a_src2, a_dst2)


def _tc_mid_body(x_ref, c0_ref, c1_ref, ftab_ref, hw1_ref, w2_ref, b1_ref,
                 vs2_ref, vd2_ref, g2_ref, asrc_ref, adst_ref):
    xb = x_ref[...]
    iot = lax.broadcasted_iota(jnp.int32, (NB, C), 1)
    oh = (xb == iot).astype(jnp.float32)
    cb = c0_ref[...] + c1_ref[...]
    dn = (((1,), (1,)), ((), ()))
    acc = jnp.zeros((NB, HID), jnp.float32)
    for h in range(H):
        fx = lax.dot_general(oh, ftab_ref[h], dn,
                             preferred_element_type=jnp.float32)
        sh = cb * fx
        den = jnp.sum(sh, axis=1, keepdims=True)
        sn = sh / (den + 1e-16)
        acc = acc + jnp.dot(sn, hw1_ref[:, h * HID:(h + 1) * HID],
                            preferred_element_type=jnp.float32)
    out1 = jnp.maximum(acc * (1.0 / H) + b1_ref[...], 0.0)
    g2 = jnp.dot(out1, w2_ref[...], preferred_element_type=jnp.float32)
    for h in range(H):
        g2_ref[h] = g2[:, h * C:(h + 1) * C]
    dnT = (((0,), (1,)), ((), ()))
    asrc_ref[...] = lax.dot_general(vs2_ref[...], out1, dnT,
                                    preferred_element_type=jnp.float32)
    adst_ref[...] = lax.dot_general(vd2_ref[...], out1, dnT,
                                    preferred_element_type=jnp.float32)


def _tc_mid(xp, C0, C1, ftab, hw1, W2, b1r, vs2, vd2):
    grid = NP // NB
    full = lambda *shape: pl.BlockSpec(shape, lambda i: (0,) * len(shape))
    return pl.pallas_call(
        _tc_mid_body,
        grid=(grid,),
        in_specs=[
            pl.BlockSpec((NB, 1), lambda i: (i, 0)),
            pl.BlockSpec((NB, C), lambda i: (i, 0)),
            pl.BlockSpec((NB, C), lambda i: (i, 0)),
            full(H, C, C),
            full(C, H * HID),
            full(HID, H * C),
            full(1, HID),
            full(HID, H),
            full(HID, H),
        ],
        out_specs=[
            pl.BlockSpec((H, NB, C), lambda i: (0, i, 0)),
            pl.BlockSpec((H, NB), lambda i: (0, i)),
            pl.BlockSpec((H, NB), lambda i: (0, i)),
        ],
        out_shape=[
            jax.ShapeDtypeStruct((H, NP, C), jnp.float32),
            jax.ShapeDtypeStruct((H, NP), jnp.float32),
            jax.ShapeDtypeStruct((H, NP), jnp.float32),
        ],
    )(xp, C0, C1, ftab, hw1, W2, b1r, vs2, vd2)


def _tc_final_body(ph_ref, den_ref, b2_ref, out_ref):
    acc = jnp.zeros((NB, C), jnp.float32)
    for h in range(H):
        dh = den_ref[:, h:h + 1]
        acc = acc + ph_ref[h] / (dh + 1e-16)
    out_ref[...] = acc * (1.0 / H) + b2_ref[...]


def _tc_final(Ph, denT, b2r):
    grid = NP // NB
    return pl.pallas_call(
        _tc_final_body,
        grid=(grid,),
        in_specs=[
            pl.BlockSpec((H, NB, PW), lambda i: (0, i, 0)),
            pl.BlockSpec((NB, H), lambda i: (i, 0)),
            pl.BlockSpec((1, C), lambda i: (0, 0)),
        ],
        out_specs=pl.BlockSpec((NB, C), lambda i: (i, 0)),
        out_shape=jax.ShapeDtypeStruct((NP, C), jnp.float32),
    )(Ph, denT, b2r)


def kernel(x, edge_index, emb, W1, a_src1, a_dst1, b1, W2, a_src2, a_dst2, b2):
    x = x.astype(jnp.int32)
    ei = edge_index.astype(jnp.int32)

    src_a = ei[0]
    dst_a = ei[1]
    cpart = _sc_hist(x, src_a, dst_a)
    ftab, hw1, vs2, vd2 = _tc_tables(
        _f32(emb), _f32(W1), _f32(a_src1), _f32(a_dst1),
        _f32(W2), _f32(a_src2), _f32(a_dst2))

    xp = jnp.pad(x, (0, NP - N)).reshape(NP, 1)
    C0 = cpart[0].reshape(NP, C)
    C1 = cpart[1].reshape(NP, C)
    b1r = _f32(b1).reshape(1, HID)
    g2T, asrcT, adstT = _tc_mid(xp, C0, C1, ftab, hw1, _f32(W2), b1r, vs2, vd2)

    g2flat = g2T.reshape(H * NP, C)
    Ph, den = _sc_l2(src_a, dst_a, g2flat,
                     asrcT.reshape(-1), adstT.reshape(-1))
    Ph = Ph.reshape(H, NP, PW)
    denT = den.reshape(H, NP).T

    out = _tc_final(Ph, denT, _f32(b2).reshape(1, C))
    return out[:N]


# speedup vs baseline: 50.6762x; 3.6654x over previous
import functools

import jax
import jax.numpy as jnp
from jax import lax
from jax.experimental import pallas as pl
from jax.experimental.pallas import tpu as pltpu
from jax.experimental.pallas import tpu_sc as plsc

N = 10000
E = 160000
C = 128
H = 8
HID = 128
DM = 256
NP = 10240
NB = 1024
NSC = 2
NT = 16
PW = 128

_MESH = plsc.VectorSubcoreMesh(core_axis_name="c", subcore_axis_name="s")


def _f32(x):
    return x.astype(jnp.float32)


EPT1 = E // (NSC * NT)
_HROWS = NP * C
_HSTRIPE = _HROWS // NT
_ZW1 = 4096


@functools.partial(
    pl.kernel,
    out_type=jax.ShapeDtypeStruct((NSC, _HROWS), jnp.float32),
    mesh=_MESH,
    compiler_params=pltpu.CompilerParams(needs_layout_passes=False),
    scratch_types=[
        pltpu.VMEM((N,), jnp.int32),
        pltpu.VMEM((EPT1,), jnp.int32),
        pltpu.VMEM((EPT1,), jnp.int32),
        pltpu.VMEM((128,), jnp.int32),
        pltpu.VMEM((128,), jnp.float32),
        pltpu.VMEM((16,), jnp.int32),
        pltpu.VMEM((16,), jnp.float32),
        pltpu.VMEM((_ZW1,), jnp.float32),
        pltpu.VMEM_SHARED((_HROWS,), jnp.float32),
    ],
)
def _sc_hist(x_hbm, src_hbm, dst_hbm, out_hbm, xv, srcv, dstv, keyb, valb,
             keyt, valt, zb, csp):
    c = lax.axis_index("c")
    s = lax.axis_index("s")
    est = (c * NT + s) * EPT1
    pltpu.sync_copy(src_hbm.at[pl.ds(est, EPT1)], srcv)
    pltpu.sync_copy(dst_hbm.at[pl.ds(est, EPT1)], dstv)
    pltpu.sync_copy(x_hbm, xv)

    zero16 = jnp.zeros((16,), jnp.float32)
    one16 = jnp.full((16,), 1.0, jnp.float32)

    def zfill(i, _):
        zb[pl.ds(i * 16, 16)] = zero16
        return 0
    lax.fori_loop(0, _ZW1 // 16, zfill, 0)

    def ofill(i, _):
        valb[pl.ds(i * 16, 16)] = one16
        return 0
    lax.fori_loop(0, 8, ofill, 0)

    def zcopy(i, _):
        pltpu.sync_copy(zb, csp.at[pl.ds(s * _HSTRIPE + i * _ZW1, _ZW1)])
        return 0
    lax.fori_loop(0, _HSTRIPE // _ZW1, zcopy, 0)
    plsc.subcore_barrier()

    def chunk(ch, _):
        def grp(g, _):
            off = ch * 128 + g * 16
            s16 = srcv[pl.ds(off, 16)]
            d16 = dstv[pl.ds(off, 16)]
            cls = plsc.load_gather(xv, [s16])
            keyb[pl.ds(g * 16, 16)] = d16 * C + cls
            return 0
        lax.fori_loop(0, 8, grp, 0)
        pltpu.sync_copy(valb, csp.at[keyb], add=True)
        return 0
    lax.fori_loop(0, 39, chunk, 0)

    off = EPT1 - 16
    s16 = srcv[pl.ds(off, 16)]
    d16 = dstv[pl.ds(off, 16)]
    cls = plsc.load_gather(xv, [s16])
    lane = lax.iota(jnp.int32, 16)
    keyt[...] = jnp.where(lane >= 8, d16 * C + cls, 0)
    valt[...] = jnp.where(lane >= 8, 1.0, 0.0)
    pltpu.sync_copy(valt, csp.at[keyt], add=True)

    plsc.subcore_barrier()
    pltpu.sync_copy(csp.at[pl.ds(s * _HSTRIPE, _HSTRIPE)],
                    out_hbm.at[c, pl.ds(s * _HSTRIPE, _HSTRIPE)])


EPT2 = E // NT
HP = H // NSC
_K2 = 80
_NCH = EPT2 // _K2
_RPT = NP // NT
_ZR2 = 32
_DRAIN_NOW = True


@functools.partial(
    pl.kernel,
    out_type=[jax.ShapeDtypeStruct((H * NP, PW), jnp.float32),
              jax.ShapeDtypeStruct((H * NP,), jnp.float32)],
    mesh=_MESH,
    compiler_params=pltpu.CompilerParams(needs_layout_passes=False),
    scratch_types=[
        [pltpu.VMEM((2 * _K2,), jnp.int32)] * 3,
        [pltpu.VMEM((_K2,), jnp.int32)] * 3,
        [pltpu.VMEM((_K2,), jnp.int32)] * 3,
        [pltpu.VMEM((_K2,), jnp.int32)] * 3,
        [pltpu.VMEM((_K2,), jnp.float32)] * 3,
        [pltpu.VMEM((_K2,), jnp.float32)] * 3,
        [pltpu.VMEM((_K2,), jnp.float32)] * 3,
        [pltpu.VMEM((_K2, PW), jnp.float32)] * 3,
        pltpu.VMEM((_ZR2, PW), jnp.float32),
        pltpu.VMEM((_RPT,), jnp.float32),
        pltpu.VMEM_SHARED((NP, PW), jnp.float32),
        pltpu.VMEM_SHARED((NP,), jnp.float32),
        [pltpu.SemaphoreType.DMA] * 3,
        [pltpu.SemaphoreType.DMA] * 3,
        [pltpu.SemaphoreType.DMA] * 3,
        [pltpu.SemaphoreType.DMA] * 3,
        [pltpu.SemaphoreType.DMA] * 3,
        pltpu.SemaphoreType.DMA,
    ],
)
def _sc_l2(sdp_hbm, g2_hbm, asrc_hbm, adst_hbm, out_hbm, den_hbm,
           sdb, idxb, idxd, dstb, exb, asv, adv, grow, zb, zbd, psp, dsp,
           semsd, semg, sega, sems, semd, semz):
    c = lax.axis_index("c")
    s = lax.axis_index("s")

    zero16 = jnp.zeros((16,), jnp.float32)

    def zfill_row(i, _):
        def zcol(j, _):
            zb[i, pl.ds(j * 16, 16)] = zero16
            return 0
        lax.fori_loop(0, PW // 16, zcol, 0)
        return 0
    lax.fori_loop(0, _ZR2, zfill_row, 0)

    def zfill_d(i, _):
        zbd[pl.ds(i * 16, 16)] = zero16
        return 0
    lax.fori_loop(0, _RPT // 16, zfill_d, 0)

    for p in range(HP):
        h = c * HP + p
        hoff = h * NP

        def zcopy(i, _):
            pltpu.async_copy(zb, psp.at[pl.ds(s * _RPT + i * _ZR2, _ZR2)],
                             semz)
            return 0
        lax.fori_loop(0, _RPT // _ZR2, zcopy, 0)
        pltpu.async_copy(zbd, dsp.at[pl.ds(s * _RPT, _RPT)], semz)

        def zdrain(i, _):
            pltpu.make_async_copy(
                zb, psp.at[pl.ds(s * _RPT, _ZR2)], semz).wait()
            return 0
        lax.fori_loop(0, _RPT // _ZR2, zdrain, 0)
        pltpu.make_async_copy(zbd, dsp.at[pl.ds(s * _RPT, _RPT)], semz).wait()
        plsc.subcore_barrier()

        def sd_issue(ch, B):
            pltpu.async_copy(
                sdp_hbm.at[pl.ds((s * _NCH + ch) * (2 * _K2), 2 * _K2)],
                sdb[B], semsd[B])

        def fill(ch, B, prefetch=True):
            pltpu.make_async_copy(
                sdp_hbm.at[pl.ds(0, 2 * _K2)], sdb[B], semsd[B]).wait()
            for g in range(_K2 // 16):
                s16 = sdb[B][pl.ds(g * 16, 16)]
                d16 = sdb[B][pl.ds(_K2 + g * 16, 16)]
                idxb[B][pl.ds(g * 16, 16)] = s16 + hoff
                idxd[B][pl.ds(g * 16, 16)] = d16 + hoff
                dstb[B][pl.ds(g * 16, 16)] = d16
            pltpu.async_copy(g2_hbm.at[idxb[B]], grow[B], semg[B])
            pltpu.async_copy(asrc_hbm.at[idxb[B]], asv[B], sega[B])
            pltpu.async_copy(adst_hbm.at[idxd[B]], adv[B], sega[B])

            if prefetch:
                @pl.when(ch + 3 < _NCH)
                def _():
                    sd_issue(ch + 3, B)

        def drain_sc(B):
            pltpu.make_async_copy(grow[B], psp.at[dstb[B]], sems[B]).wait()
            pltpu.make_async_copy(exb[B], dsp.at[dstb[B]], semd[B]).wait()

        def proc(B):

            pltpu.make_async_copy(g2_hbm.at[idxb[B]], grow[B], semg[B]).wait()
            pltpu.make_async_copy(asrc_hbm.at[idxb[B]], asv[B], sega[B]).wait()
            pltpu.make_async_copy(adst_hbm.at[idxd[B]], adv[B], sega[B]).wait()
            for g in range(_K2 // 16):
                ea = asv[B][pl.ds(g * 16, 16)] + adv[B][pl.ds(g * 16, 16)]
                ea = jnp.where(ea >= 0.0, ea, 0.2 * ea)
                exb[B][pl.ds(g * 16, 16)] = jnp.exp(ea)

            def sc5(k, _):
                for j in range(5):
                    i = k * 5 + j
                    exs = plsc.load_gather(
                        exb[B], [jnp.full((16,), i, jnp.int32)])
                    for t in range(PW // 16):
                        grow[B][i, pl.ds(t * 16, 16)] = (
                            grow[B][i, pl.ds(t * 16, 16)] * exs)
                return 0
            lax.fori_loop(0, _K2 // 5, sc5, 0)
            pltpu.async_copy(grow[B], psp.at[dstb[B]], sems[B], add=True)
            pltpu.async_copy(exb[B], dsp.at[dstb[B]], semd[B], add=True)
            if _DRAIN_NOW:
                drain_sc(B)

        for B in range(3):
            sd_issue(B, B)
        fill(0, 0)
        fill(1, 1)

        def triple(i, _):
            for jj, B in ((0, 2), (1, 0), (2, 1)):
                ch = 3 * i + 2 + jj
                proc((B + 1) % 3)
                if not _DRAIN_NOW:
                    if jj == 0:
                        @pl.when(ch >= 3)
                        def _():
                            drain_sc(B)
                    else:
                        drain_sc(B)
                fill(ch, B)
            return 0
        lax.fori_loop(0, 41, triple, 0)

        proc(0)
        proc(1)
        if not _DRAIN_NOW:
            for B in range(3):
                drain_sc(B)
        plsc.subcore_barrier()

        pltpu.sync_copy(psp.at[pl.ds(s * _RPT, _RPT)],
                        out_hbm.at[pl.ds(hoff + s * _RPT, _RPT)])
        pltpu.sync_copy(dsp.at[pl.ds(s * _RPT, _RPT)],
                        den_hbm.at[pl.ds(hoff + s * _RPT, _RPT)])


def _tc_tables_body(emb_ref, w1_ref, as1_ref, ad1_ref, w2_ref, as2_ref,
                    ad2_ref, ftab_ref, hw1_ref, vs2_ref, vd2_ref):
    hw = jnp.dot(emb_ref[...], w1_ref[...], preferred_element_type=jnp.float32)
    hw1_ref[...] = hw
    dn = (((1,), (1,)), ((), ()))
    for h in range(H):
        hwh = hw[:, h * HID:(h + 1) * HID]
        a1c = lax.dot_general(hwh, as1_ref[h:h + 1, :], dn,
                              preferred_element_type=jnp.float32)
        b1r = lax.dot_general(ad1_ref[h:h + 1, :], hwh, dn,
                              preferred_element_type=jnp.float32)
        e = a1c + b1r
        ftab_ref[h] = jnp.exp(jnp.where(e >= 0.0, e, 0.2 * e))
        w2h = w2_ref[:, h * C:(h + 1) * C]
        vs2_ref[:, h:h + 1] = lax.dot_general(
            w2h, as2_ref[h:h + 1, :], dn, preferred_element_type=jnp.float32)
        vd2_ref[:, h:h + 1] = lax.dot_general(
            w2h, ad2_ref[h:h + 1, :], dn, preferred_element_type=jnp.float32)


def _tc_tables(emb, W1, a_src1, a_dst1, W2, a_src2, a_dst2):
    return pl.pallas_call(
        _tc_tables_body,
        out_shape=[
            jax.ShapeDtypeStruct((H, C, C), jnp.float32),
            jax.ShapeDtypeStruct((C, H * HID), jnp.float32),
            jax.ShapeDtypeStruct((HID, H), jnp.float32),
            jax.ShapeDtypeStruct((HID, H), jnp.float32),
        ],
    )(emb, W1, a_src1, a_dst1, W2, a_src2, a_dst2)


def _tc_mid_body(x_ref, c0_ref, c1_ref, ftab_ref, hw1_ref, w2_ref, b1_ref,
                 vs2_ref, vd2_ref, g2_ref, asrc_ref, adst_ref):
    xb = x_ref[...]
    iot = lax.broadcasted_iota(jnp.int32, (NB, C), 1)
    oh = (xb == iot).astype(jnp.float32)
    cb = c0_ref[...] + c1_ref[...]
    dn = (((1,), (1,)), ((), ()))
    acc = jnp.zeros((NB, HID), jnp.float32)
    for h in range(H):
        fx = lax.dot_general(oh, ftab_ref[h], dn,
                             preferred_element_type=jnp.float32)
        sh = cb * fx
        den = jnp.sum(sh, axis=1, keepdims=True)
        sn = sh / (den + 1e-16)
        acc = acc + jnp.dot(sn, hw1_ref[:, h * HID:(h + 1) * HID],
                            preferred_element_type=jnp.float32)
    out1 = jnp.maximum(acc * (1.0 / H) + b1_ref[...], 0.0)
    g2 = jnp.dot(out1, w2_ref[...], preferred_element_type=jnp.float32)
    for h in range(H):
        g2_ref[h] = g2[:, h * C:(h + 1) * C]
    dnT = (((0,), (1,)), ((), ()))
    asrc_ref[...] = lax.dot_general(vs2_ref[...], out1, dnT,
                                    preferred_element_type=jnp.float32)
    adst_ref[...] = lax.dot_general(vd2_ref[...], out1, dnT,
                                    preferred_element_type=jnp.float32)


def _tc_mid(xp, C0, C1, ftab, hw1, W2, b1r, vs2, vd2):
    grid = NP // NB
    full = lambda *shape: pl.BlockSpec(shape, lambda i: (0,) * len(shape))
    return pl.pallas_call(
        _tc_mid_body,
        grid=(grid,),
        in_specs=[
            pl.BlockSpec((NB, 1), lambda i: (i, 0)),
            pl.BlockSpec((NB, C), lambda i: (i, 0)),
            pl.BlockSpec((NB, C), lambda i: (i, 0)),
            full(H, C, C),
            full(C, H * HID),
            full(HID, H * C),
            full(1, HID),
            full(HID, H),
            full(HID, H),
        ],
        out_specs=[
            pl.BlockSpec((H, NB, C), lambda i: (0, i, 0)),
            pl.BlockSpec((H, NB), lambda i: (0, i)),
            pl.BlockSpec((H, NB), lambda i: (0, i)),
        ],
        out_shape=[
            jax.ShapeDtypeStruct((H, NP, C), jnp.float32),
            jax.ShapeDtypeStruct((H, NP), jnp.float32),
            jax.ShapeDtypeStruct((H, NP), jnp.float32),
        ],
    )(xp, C0, C1, ftab, hw1, W2, b1r, vs2, vd2)


def _tc_final_body(ph_ref, den_ref, b2_ref, out_ref):
    acc = jnp.zeros((NB, C), jnp.float32)
    for h in range(H):
        dh = den_ref[:, h:h + 1]
        acc = acc + ph_ref[h] / (dh + 1e-16)
    out_ref[...] = acc * (1.0 / H) + b2_ref[...]


def _tc_final(Ph, denT, b2r):
    grid = NP // NB
    return pl.pallas_call(
        _tc_final_body,
        grid=(grid,),
        in_specs=[
            pl.BlockSpec((H, NB, PW), lambda i: (0, i, 0)),
            pl.BlockSpec((NB, H), lambda i: (i, 0)),
            pl.BlockSpec((1, C), lambda i: (0, 0)),
        ],
        out_specs=pl.BlockSpec((NB, C), lambda i: (i, 0)),
        out_shape=jax.ShapeDtypeStruct((NP, C), jnp.float32),
    )(Ph, denT, b2r)


def kernel(x, edge_index, emb, W1, a_src1, a_dst1, b1, W2, a_src2, a_dst2, b2):
    x = x.astype(jnp.int32)
    ei = edge_index.astype(jnp.int32)

    src_a = ei[0]
    dst_a = ei[1]
    cpart = _sc_hist(x, src_a, dst_a)
    srcr = src_a.reshape(E // _K2, _K2)
    dstr = dst_a.reshape(E // _K2, _K2)
    sdp = jnp.concatenate([srcr, dstr], axis=1).reshape(-1)
    ftab, hw1, vs2, vd2 = _tc_tables(
        _f32(emb), _f32(W1), _f32(a_src1), _f32(a_dst1),
        _f32(W2), _f32(a_src2), _f32(a_dst2))

    xp = jnp.pad(x, (0, NP - N)).reshape(NP, 1)
    C0 = cpart[0].reshape(NP, C)
    C1 = cpart[1].reshape(NP, C)
    b1r = _f32(b1).reshape(1, HID)
    g2T, asrcT, adstT = _tc_mid(xp, C0, C1, ftab, hw1, _f32(W2), b1r, vs2, vd2)

    g2flat = g2T.reshape(H * NP, C)
    Ph, den = _sc_l2(sdp, g2flat, asrcT.reshape(-1), adstT.reshape(-1))
    Ph = Ph.reshape(H, NP, PW)
    denT = den.reshape(H, NP).T

    out = _tc_final(Ph, denT, _f32(b2).reshape(1, C))
    return out[:N]


# speedup vs baseline: 58.8085x; 1.1605x over previous
import functools

import jax
import jax.numpy as jnp
from jax import lax
from jax.experimental import pallas as pl
from jax.experimental.pallas import tpu as pltpu
from jax.experimental.pallas import tpu_sc as plsc

N = 10000
E = 160000
C = 128
H = 8
HID = 128
DM = 256
NP = 10240
NB = 1024
NSC = 2
NT = 16
PW = 128

_MESH = plsc.VectorSubcoreMesh(core_axis_name="c", subcore_axis_name="s")


def _f32(x):
    return x.astype(jnp.float32)


EPT1 = E // (NSC * NT)
_HROWS = NP * C
_HSTRIPE = _HROWS // NT
_ZW1 = 4096


@functools.partial(
    pl.kernel,
    out_type=jax.ShapeDtypeStruct((NSC, _HROWS), jnp.float32),
    mesh=_MESH,
    compiler_params=pltpu.CompilerParams(needs_layout_passes=False),
    scratch_types=[
        pltpu.VMEM((N,), jnp.int32),
        pltpu.VMEM((EPT1,), jnp.int32),
        pltpu.VMEM((EPT1,), jnp.int32),
        pltpu.VMEM((128,), jnp.int32),
        pltpu.VMEM((128,), jnp.float32),
        pltpu.VMEM((16,), jnp.int32),
        pltpu.VMEM((16,), jnp.float32),
        pltpu.VMEM((_ZW1,), jnp.float32),
        pltpu.VMEM_SHARED((_HROWS,), jnp.float32),
    ],
)
def _sc_hist(x_hbm, src_hbm, dst_hbm, out_hbm, xv, srcv, dstv, keyb, valb,
             keyt, valt, zb, csp):
    c = lax.axis_index("c")
    s = lax.axis_index("s")
    est = (c * NT + s) * EPT1
    pltpu.sync_copy(src_hbm.at[pl.ds(est, EPT1)], srcv)
    pltpu.sync_copy(dst_hbm.at[pl.ds(est, EPT1)], dstv)
    pltpu.sync_copy(x_hbm, xv)

    zero16 = jnp.zeros((16,), jnp.float32)
    one16 = jnp.full((16,), 1.0, jnp.float32)

    def zfill(i, _):
        zb[pl.ds(i * 16, 16)] = zero16
        return 0
    lax.fori_loop(0, _ZW1 // 16, zfill, 0)

    def ofill(i, _):
        valb[pl.ds(i * 16, 16)] = one16
        return 0
    lax.fori_loop(0, 8, ofill, 0)

    def zcopy(i, _):
        pltpu.sync_copy(zb, csp.at[pl.ds(s * _HSTRIPE + i * _ZW1, _ZW1)])
        return 0
    lax.fori_loop(0, _HSTRIPE // _ZW1, zcopy, 0)
    plsc.subcore_barrier()

    def chunk(ch, _):
        def grp(g, _):
            off = ch * 128 + g * 16
            s16 = srcv[pl.ds(off, 16)]
            d16 = dstv[pl.ds(off, 16)]
            cls = plsc.load_gather(xv, [s16])
            keyb[pl.ds(g * 16, 16)] = d16 * C + cls
            return 0
        lax.fori_loop(0, 8, grp, 0)
        pltpu.sync_copy(valb, csp.at[keyb], add=True)
        return 0
    lax.fori_loop(0, 39, chunk, 0)

    off = EPT1 - 16
    s16 = srcv[pl.ds(off, 16)]
    d16 = dstv[pl.ds(off, 16)]
    cls = plsc.load_gather(xv, [s16])
    lane = lax.iota(jnp.int32, 16)
    keyt[...] = jnp.where(lane >= 8, d16 * C + cls, 0)
    valt[...] = jnp.where(lane >= 8, 1.0, 0.0)
    pltpu.sync_copy(valt, csp.at[keyt], add=True)

    plsc.subcore_barrier()
    pltpu.sync_copy(csp.at[pl.ds(s * _HSTRIPE, _HSTRIPE)],
                    out_hbm.at[c, pl.ds(s * _HSTRIPE, _HSTRIPE)])


EPT2 = E // NT
HP = H // NSC
_K2 = 80
_NCH = EPT2 // _K2
_RPT = NP // NT
_ZR2 = 32
_DRAIN_NOW = False


@functools.partial(
    pl.kernel,
    out_type=[jax.ShapeDtypeStruct((H * NP, PW), jnp.float32),
              jax.ShapeDtypeStruct((H * NP,), jnp.float32)],
    mesh=_MESH,
    compiler_params=pltpu.CompilerParams(needs_layout_passes=False),
    scratch_types=[
        [pltpu.VMEM((2 * _K2,), jnp.int32)] * 3,
        [pltpu.VMEM((_K2,), jnp.int32)] * 3,
        [pltpu.VMEM((_K2,), jnp.int32)] * 3,
        [pltpu.VMEM((_K2,), jnp.int32)] * 3,
        [pltpu.VMEM((_K2,), jnp.float32)] * 3,
        [pltpu.VMEM((_K2,), jnp.float32)] * 3,
        [pltpu.VMEM((_K2,), jnp.float32)] * 3,
        [pltpu.VMEM((_K2, PW), jnp.float32)] * 3,
        pltpu.VMEM((_ZR2, PW), jnp.float32),
        pltpu.VMEM((_RPT,), jnp.float32),
        pltpu.VMEM_SHARED((NP, PW), jnp.float32),
        pltpu.VMEM_SHARED((NP,), jnp.float32),
        [pltpu.SemaphoreType.DMA] * 3,
        [pltpu.SemaphoreType.DMA] * 3,
        [pltpu.SemaphoreType.DMA] * 3,
        [pltpu.SemaphoreType.DMA] * 3,
        [pltpu.SemaphoreType.DMA] * 3,
        pltpu.SemaphoreType.DMA,
    ],
)
def _sc_l2(sdp_hbm, g2_hbm, asrc_hbm, adst_hbm, out_hbm, den_hbm,
           sdb, idxb, idxd, dstb, exb, asv, adv, grow, zb, zbd, psp, dsp,
           semsd, semg, sega, sems, semd, semz):
    c = lax.axis_index("c")
    s = lax.axis_index("s")

    zero16 = jnp.zeros((16,), jnp.float32)

    def zfill_row(i, _):
        def zcol(j, _):
            zb[i, pl.ds(j * 16, 16)] = zero16
            return 0
        lax.fori_loop(0, PW // 16, zcol, 0)
        return 0
    lax.fori_loop(0, _ZR2, zfill_row, 0)

    def zfill_d(i, _):
        zbd[pl.ds(i * 16, 16)] = zero16
        return 0
    lax.fori_loop(0, _RPT // 16, zfill_d, 0)

    for p in range(HP):
        h = c * HP + p
        hoff = h * NP

        def zcopy(i, _):
            pltpu.async_copy(zb, psp.at[pl.ds(s * _RPT + i * _ZR2, _ZR2)],
                             semz)
            return 0
        lax.fori_loop(0, _RPT // _ZR2, zcopy, 0)
        pltpu.async_copy(zbd, dsp.at[pl.ds(s * _RPT, _RPT)], semz)

        def zdrain(i, _):
            pltpu.make_async_copy(
                zb, psp.at[pl.ds(s * _RPT, _ZR2)], semz).wait()
            return 0
        lax.fori_loop(0, _RPT // _ZR2, zdrain, 0)
        pltpu.make_async_copy(zbd, dsp.at[pl.ds(s * _RPT, _RPT)], semz).wait()
        plsc.subcore_barrier()

        def sd_issue(ch, B):
            pltpu.async_copy(
                sdp_hbm.at[pl.ds((s * _NCH + ch) * (2 * _K2), 2 * _K2)],
                sdb[B], semsd[B])

        def fill(ch, B, prefetch=True):
            pltpu.make_async_copy(
                sdp_hbm.at[pl.ds(0, 2 * _K2)], sdb[B], semsd[B]).wait()
            for g in range(_K2 // 16):
                s16 = sdb[B][pl.ds(g * 16, 16)]
                d16 = sdb[B][pl.ds(_K2 + g * 16, 16)]
                idxb[B][pl.ds(g * 16, 16)] = s16 + hoff
                idxd[B][pl.ds(g * 16, 16)] = d16 + hoff
                dstb[B][pl.ds(g * 16, 16)] = d16
            pltpu.async_copy(g2_hbm.at[idxb[B]], grow[B], semg[B])
            pltpu.async_copy(asrc_hbm.at[idxb[B]], asv[B], sega[B])
            pltpu.async_copy(adst_hbm.at[idxd[B]], adv[B], sega[B])

            if prefetch:
                @pl.when(ch + 3 < _NCH)
                def _():
                    sd_issue(ch + 3, B)

        def drain_sc(B):
            pltpu.make_async_copy(grow[B], psp.at[dstb[B]], sems[B]).wait()
            pltpu.make_async_copy(exb[B], dsp.at[dstb[B]], semd[B]).wait()

        def proc(B, drainB=None, gate=None):

            pltpu.make_async_copy(g2_hbm.at[idxb[B]], grow[B], semg[B]).wait()
            pltpu.make_async_copy(asrc_hbm.at[idxb[B]], asv[B], sega[B]).wait()
            pltpu.make_async_copy(adst_hbm.at[idxd[B]], adv[B], sega[B]).wait()
            for g in range(_K2 // 16):
                ea = asv[B][pl.ds(g * 16, 16)] + adv[B][pl.ds(g * 16, 16)]
                ea = jnp.where(ea >= 0.0, ea, 0.2 * ea)
                exb[B][pl.ds(g * 16, 16)] = jnp.exp(ea)

            def sc5(k, _):
                for j in range(5):
                    i = k * 5 + j
                    exs = plsc.load_gather(
                        exb[B], [jnp.full((16,), i, jnp.int32)])
                    for t in range(PW // 16):
                        grow[B][i, pl.ds(t * 16, 16)] = (
                            grow[B][i, pl.ds(t * 16, 16)] * exs)
                return 0
            lax.fori_loop(0, _K2 // 5, sc5, 0)
            if drainB is not None:
                if gate is None:
                    drain_sc(drainB)
                else:
                    @pl.when(gate)
                    def _():
                        drain_sc(drainB)
            pltpu.async_copy(grow[B], psp.at[dstb[B]], sems[B], add=True)
            pltpu.async_copy(exb[B], dsp.at[dstb[B]], semd[B], add=True)
            if _DRAIN_NOW:
                drain_sc(B)

        for B in range(3):
            sd_issue(B, B)
        fill(0, 0)
        fill(1, 1)

        def triple(i, _):
            for jj, B in ((0, 2), (1, 0), (2, 1)):
                ch = 3 * i + 2 + jj
                proc((B + 1) % 3, drainB=B,
                     gate=(ch >= 3) if jj == 0 else None)
                fill(ch, B)
            return 0
        lax.fori_loop(0, 41, triple, 0)

        proc(0, drainB=2)
        proc(1, drainB=0)
        drain_sc(1)
        plsc.subcore_barrier()

        pltpu.sync_copy(psp.at[pl.ds(s * _RPT, _RPT)],
                        out_hbm.at[pl.ds(hoff + s * _RPT, _RPT)])
        pltpu.sync_copy(dsp.at[pl.ds(s * _RPT, _RPT)],
                        den_hbm.at[pl.ds(hoff + s * _RPT, _RPT)])


def _tc_tables_body(emb_ref, w1_ref, as1_ref, ad1_ref, w2_ref, as2_ref,
                    ad2_ref, ftab_ref, hw1_ref, vs2_ref, vd2_ref):
    hw = jnp.dot(emb_ref[...], w1_ref[...], preferred_element_type=jnp.float32)
    hw1_ref[...] = hw
    dn = (((1,), (1,)), ((), ()))
    for h in range(H):
        hwh = hw[:, h * HID:(h + 1) * HID]
        a1c = lax.dot_general(hwh, as1_ref[h:h + 1, :], dn,
                              preferred_element_type=jnp.float32)
        b1r = lax.dot_general(ad1_ref[h:h + 1, :], hwh, dn,
                              preferred_element_type=jnp.float32)
        e = a1c + b1r
        ftab_ref[h] = jnp.exp(jnp.where(e >= 0.0, e, 0.2 * e))
        w2h = w2_ref[:, h * C:(h + 1) * C]
        vs2_ref[:, h:h + 1] = lax.dot_general(
            w2h, as2_ref[h:h + 1, :], dn, preferred_element_type=jnp.float32)
        vd2_ref[:, h:h + 1] = lax.dot_general(
            w2h, ad2_ref[h:h + 1, :], dn, preferred_element_type=jnp.float32)


def _tc_tables(emb, W1, a_src1, a_dst1, W2, a_src2, a_dst2):
    return pl.pallas_call(
        _tc_tables_body,
        out_shape=[
            jax.ShapeDtypeStruct((H, C, C), jnp.float32),
            jax.ShapeDtypeStruct((C, H * HID), jnp.float32),
            jax.ShapeDtypeStruct((HID, H), jnp.float32),
            jax.ShapeDtypeStruct((HID, H), jnp.float32),
        ],
    )(emb, W1, a_src1, a_dst1, W2, a_src2, a_dst2)


def _tc_mid_body(x_ref, c0_ref, c1_ref, ftab_ref, hw1_ref, w2_ref, b1_ref,
                 vs2_ref, vd2_ref, g2_ref, asrc_ref, adst_ref):
    xb = x_ref[...]
    iot = lax.broadcasted_iota(jnp.int32, (NB, C), 1)
    oh = (xb == iot).astype(jnp.float32)
    cb = c0_ref[...] + c1_ref[...]
    dn = (((1,), (1,)), ((), ()))
    acc = jnp.zeros((NB, HID), jnp.float32)
    for h in range(H):
        fx = lax.dot_general(oh, ftab_ref[h], dn,
                             preferred_element_type=jnp.float32)
        sh = cb * fx
        den = jnp.sum(sh, axis=1, keepdims=True)
        sn = sh / (den + 1e-16)
        acc = acc + jnp.dot(sn, hw1_ref[:, h * HID:(h + 1) * HID],
                            preferred_element_type=jnp.float32)
    out1 = jnp.maximum(acc * (1.0 / H) + b1_ref[...], 0.0)
    g2 = jnp.dot(out1, w2_ref[...], preferred_element_type=jnp.float32)
    for h in range(H):
        g2_ref[h] = g2[:, h * C:(h + 1) * C]
    dnT = (((0,), (1,)), ((), ()))
    asrc_ref[...] = lax.dot_general(vs2_ref[...], out1, dnT,
                                    preferred_element_type=jnp.float32)
    adst_ref[...] = lax.dot_general(vd2_ref[...], out1, dnT,
                                    preferred_element_type=jnp.float32)


def _tc_mid(xp, C0, C1, ftab, hw1, W2, b1r, vs2, vd2):
    grid = NP // NB
    full = lambda *shape: pl.BlockSpec(shape, lambda i: (0,) * len(shape))
    return pl.pallas_call(
        _tc_mid_body,
        grid=(grid,),
        in_specs=[
            pl.BlockSpec((NB, 1), lambda i: (i, 0)),
            pl.BlockSpec((NB, C), lambda i: (i, 0)),
            pl.BlockSpec((NB, C), lambda i: (i, 0)),
            full(H, C, C),
            full(C, H * HID),
            full(HID, H * C),
            full(1, HID),
            full(HID, H),
            full(HID, H),
        ],
        out_specs=[
            pl.BlockSpec((H, NB, C), lambda i: (0, i, 0)),
            pl.BlockSpec((H, NB), lambda i: (0, i)),
            pl.BlockSpec((H, NB), lambda i: (0, i)),
        ],
        out_shape=[
            jax.ShapeDtypeStruct((H, NP, C), jnp.float32),
            jax.ShapeDtypeStruct((H, NP), jnp.float32),
            jax.ShapeDtypeStruct((H, NP), jnp.float32),
        ],
    )(xp, C0, C1, ftab, hw1, W2, b1r, vs2, vd2)


def _tc_final_body(ph_ref, den_ref, b2_ref, out_ref):
    acc = jnp.zeros((NB, C), jnp.float32)
    for h in range(H):
        dh = den_ref[:, h:h + 1]
        acc = acc + ph_ref[h] / (dh + 1e-16)
    out_ref[...] = acc * (1.0 / H) + b2_ref[...]


def _tc_final(Ph, denT, b2r):
    grid = NP // NB
    return pl.pallas_call(
        _tc_final_body,
        grid=(grid,),
        in_specs=[
            pl.BlockSpec((H, NB, PW), lambda i: (0, i, 0)),
            pl.BlockSpec((NB, H), lambda i: (i, 0)),
            pl.BlockSpec((1, C), lambda i: (0, 0)),
        ],
        out_specs=pl.BlockSpec((NB, C), lambda i: (i, 0)),
        out_shape=jax.ShapeDtypeStruct((NP, C), jnp.float32),
    )(Ph, denT, b2r)


def kernel(x, edge_index, emb, W1, a_src1, a_dst1, b1, W2, a_src2, a_dst2, b2):
    x = x.astype(jnp.int32)
    ei = edge_index.astype(jnp.int32)

    src_a = ei[0]
    dst_a = ei[1]
    cpart = _sc_hist(x, src_a, dst_a)
    srcr = src_a.reshape(E // _K2, _K2)
    dstr = dst_a.reshape(E // _K2, _K2)
    sdp = jnp.concatenate([srcr, dstr], axis=1).reshape(-1)
    ftab, hw1, vs2, vd2 = _tc_tables(
        _f32(emb), _f32(W1), _f32(a_src1), _f32(a_dst1),
        _f32(W2), _f32(a_src2), _f32(a_dst2))

    xp = jnp.pad(x, (0, NP - N)).reshape(NP, 1)
    C0 = cpart[0].reshape(NP, C)
    C1 = cpart[1].reshape(NP, C)
    b1r = _f32(b1).reshape(1, HID)
    g2T, asrcT, adstT = _tc_mid(xp, C0, C1, ftab, hw1, _f32(W2), b1r, vs2, vd2)

    g2flat = g2T.reshape(H * NP, C)
    Ph, den = _sc_l2(sdp, g2flat, asrcT.reshape(-1), adstT.reshape(-1))
    Ph = Ph.reshape(H, NP, PW)
    denT = den.reshape(H, NP).T

    out = _tc_final(Ph, denT, _f32(b2).reshape(1, C))
    return out[:N]
